# baseline (speedup 1.0000x reference)
import functools

import jax
import jax.numpy as jnp
from jax import lax
from jax.experimental import pallas as pl
from jax.experimental.pallas import tpu as pltpu
from jax.experimental.pallas import tpu_sc as plsc

B = 16384
D = 64
L = 16
GCH = 128


def _sc_info():
    try:
        info = plsc.get_sparse_core_info()
        return info.num_cores, info.num_subcores
    except Exception:
        return 2, 16


def _body(uids_hbm, pos_hbm, neg_hbm, user_hbm, item_hbm,
          pos_out_hbm, neg_out_hbm,
          iu_v, ip_v, in_v, u_v, p_v, n_v, opos_v, oneg_v, sem,
          *, nc, bpw):
    wid = lax.axis_index("s") * nc + lax.axis_index("c")
    base = wid * bpw

    pltpu.sync_copy(uids_hbm.at[pl.ds(base, bpw)], iu_v)
    pltpu.sync_copy(pos_hbm.at[pl.ds(base, bpw)], ip_v)
    pltpu.sync_copy(neg_hbm.at[pl.ds(base, bpw)], in_v)

    copies = []
    for j in range(bpw // GCH):
        sl = pl.ds(j * GCH, GCH)
        copies.append(pltpu.async_copy(user_hbm.at[iu_v.at[sl]], u_v.at[sl], sem))
        copies.append(pltpu.async_copy(item_hbm.at[ip_v.at[sl]], p_v.at[sl], sem))
        copies.append(pltpu.async_copy(item_hbm.at[in_v.at[sl]], n_v.at[sl], sem))
    for c in copies:
        c.wait()

    lane = lax.iota(jnp.int32, L)
    masks = [lane == j for j in range(L)]

    def group(g, _):
        vp = jnp.zeros((L,), jnp.float32)
        vn = jnp.zeros((L,), jnp.float32)
        for j in range(L):
            i = g * L + j
            ap = jnp.zeros((L,), jnp.float32)
            an = jnp.zeros((L,), jnp.float32)
            for c in range(D // L):
                u = u_v[i, pl.ds(c * L, L)]
                ap = ap + u * p_v[i, pl.ds(c * L, L)]
                an = an + u * n_v[i, pl.ds(c * L, L)]
            vp = jnp.where(masks[j], jnp.sum(ap), vp)
            vn = jnp.where(masks[j], jnp.sum(an), vn)
        opos_v[pl.ds(g * L, L)] = vp
        oneg_v[pl.ds(g * L, L)] = vn
        return 0

    lax.fori_loop(0, bpw // L, group, 0)

    pltpu.sync_copy(opos_v, pos_out_hbm.at[pl.ds(base, bpw)])
    pltpu.sync_copy(oneg_v, neg_out_hbm.at[pl.ds(base, bpw)])


def kernel(uids, pos_iids, neg_iids, embed_user, embed_item):
    nc, ns = _sc_info()
    nw = nc * ns
    bpw = B // nw
    mesh = plsc.VectorSubcoreMesh(core_axis_name="c", subcore_axis_name="s")
    k = pl.kernel(
        functools.partial(_body, nc=nc, bpw=bpw),
        out_type=(
            jax.ShapeDtypeStruct((B,), jnp.float32),
            jax.ShapeDtypeStruct((B,), jnp.float32),
        ),
        mesh=mesh,
        scratch_types=[
            pltpu.VMEM((bpw,), jnp.int32),
            pltpu.VMEM((bpw,), jnp.int32),
            pltpu.VMEM((bpw,), jnp.int32),
            pltpu.VMEM((bpw, D), jnp.float32),
            pltpu.VMEM((bpw, D), jnp.float32),
            pltpu.VMEM((bpw, D), jnp.float32),
            pltpu.VMEM((bpw,), jnp.float32),
            pltpu.VMEM((bpw,), jnp.float32),
            pltpu.SemaphoreType.DMA,
        ],
        compiler_params=pltpu.CompilerParams(
            needs_layout_passes=False, use_tc_tiling_on_sc=False),
    )
    return k(uids, pos_iids, neg_iids, embed_user, embed_item)


# speedup vs baseline: 1.5642x; 1.5642x over previous
import functools

import jax
import jax.numpy as jnp
from jax import lax
from jax.experimental import pallas as pl
from jax.experimental.pallas import tpu as pltpu
from jax.experimental.pallas import tpu_sc as plsc

B = 16384
D = 64
L = 16
CH = 256


def _sc_info():
    try:
        info = plsc.get_sparse_core_info()
        return info.num_cores, info.num_subcores
    except Exception:
        return 2, 16


def _body(uids_hbm, pos_hbm, neg_hbm, user_hbm, item_hbm,
          pos_out_hbm, neg_out_hbm,
          iu_v, ip_v, in_v, u_v, p_v, n_v, opos_v, oneg_v, sem,
          *, nc, bpw):
    wid = lax.axis_index("s") * nc + lax.axis_index("c")
    base = wid * bpw

    pltpu.sync_copy(uids_hbm.at[pl.ds(base, bpw)], iu_v)
    pltpu.sync_copy(pos_hbm.at[pl.ds(base, bpw)], ip_v)
    pltpu.sync_copy(neg_hbm.at[pl.ds(base, bpw)], in_v)

    lane = lax.iota(jnp.int32, L)
    masks = [lane == j for j in range(L)]

    for ch in range(bpw // CH):
        rbase = ch * CH

        def fire(g, _):
            idu = iu_v[pl.ds(rbase + g * L, L)]
            idp = ip_v[pl.ds(rbase + g * L, L)]
            idn = in_v[pl.ds(rbase + g * L, L)]
            for j in range(L):
                r = g * L + j
                pltpu.async_copy(user_hbm.at[idu[j]], u_v.at[r], sem)
                pltpu.async_copy(item_hbm.at[idp[j]], p_v.at[r], sem)
                pltpu.async_copy(item_hbm.at[idn[j]], n_v.at[r], sem)
            return 0

        lax.fori_loop(0, CH // L, fire, 0)

        def drain(g, _):
            for j in range(L):
                r = g * L + j
                pltpu.make_async_copy(user_hbm.at[0], u_v.at[r], sem).wait()
                pltpu.make_async_copy(item_hbm.at[0], p_v.at[r], sem).wait()
                pltpu.make_async_copy(item_hbm.at[0], n_v.at[r], sem).wait()
            return 0

        lax.fori_loop(0, CH // L, drain, 0)

        def group(g, _):
            vp = jnp.zeros((L,), jnp.float32)
            vn = jnp.zeros((L,), jnp.float32)
            for j in range(L):
                r = g * L + j
                ap = jnp.zeros((L,), jnp.float32)
                an = jnp.zeros((L,), jnp.float32)
                for c in range(D // L):
                    u = u_v[r, pl.ds(c * L, L)]
                    ap = ap + u * p_v[r, pl.ds(c * L, L)]
                    an = an + u * n_v[r, pl.ds(c * L, L)]
                vp = jnp.where(masks[j], jnp.sum(ap), vp)
                vn = jnp.where(masks[j], jnp.sum(an), vn)
            opos_v[pl.ds(rbase + g * L, L)] = vp
            oneg_v[pl.ds(rbase + g * L, L)] = vn
            return 0

        lax.fori_loop(0, CH // L, group, 0)

    pltpu.sync_copy(opos_v, pos_out_hbm.at[pl.ds(base, bpw)])
    pltpu.sync_copy(oneg_v, neg_out_hbm.at[pl.ds(base, bpw)])


def kernel(uids, pos_iids, neg_iids, embed_user, embed_item):
    nc, ns = _sc_info()
    nw = nc * ns
    bpw = B // nw
    mesh = plsc.VectorSubcoreMesh(core_axis_name="c", subcore_axis_name="s")
    k = pl.kernel(
        functools.partial(_body, nc=nc, bpw=bpw),
        out_type=(
            jax.ShapeDtypeStruct((B,), jnp.float32),
            jax.ShapeDtypeStruct((B,), jnp.float32),
        ),
        mesh=mesh,
        scratch_types=[
            pltpu.VMEM((bpw,), jnp.int32),
            pltpu.VMEM((bpw,), jnp.int32),
            pltpu.VMEM((bpw,), jnp.int32),
            pltpu.VMEM((CH, D), jnp.float32),
            pltpu.VMEM((CH, D), jnp.float32),
            pltpu.VMEM((CH, D), jnp.float32),
            pltpu.VMEM((bpw,), jnp.float32),
            pltpu.VMEM((bpw,), jnp.float32),
            pltpu.SemaphoreType.DMA,
        ],
        compiler_params=pltpu.CompilerParams(needs_layout_passes=False),
    )
    return k(uids, pos_iids, neg_iids, embed_user, embed_item)
